# Initial kernel scaffold; baseline (speedup 1.0000x reference)
import jax
import jax.numpy as jnp
from jax.experimental import pallas as pl


def kernel(x, W1, b1, W2, b2, som_centroids, som_hidd_centroids):
    raise NotImplementedError("write your pallas kernel here")


# speedup vs baseline: 1.3288x; 1.3288x over previous
import functools

import jax
import jax.numpy as jnp
from jax.experimental import pallas as pl

_M, _N = 10, 10
_MH, _NH = 8, 8
_D_IN = 2576
_D_HID = 60
_D_OUT = 40
_B = 4096

_TB = 256


def _dotT(a, b):
    return jax.lax.dot_general(
        a, b, (((1,), (1,)), ((), ())), preferred_element_type=jnp.float32
    )


def _bmu_feature(scores, cmean_lane, n):
    m = jnp.min(scores, axis=1, keepdims=True)
    iota = jax.lax.broadcasted_iota(jnp.int32, scores.shape, 1)
    idx = jnp.min(jnp.where(scores == m, iota, n), axis=1, keepdims=True)
    return jnp.sum(jnp.where(iota == idx, cmean_lane, 0.0), axis=1, keepdims=True)


def _emsom_kernel(x_ref, w1_ref, b1_ref, w2_ref, b2_ref, c_ref, ch_ref,
                  out_ref, hid_ref):
    xb = x_ref[...]
    C = c_ref[...]
    CH = ch_ref[...]

    ones_d = jnp.ones((1, _D_IN), jnp.float32)
    c2 = _dotT(ones_d, C * C)
    cmean = _dotT(ones_d, C) * (1.0 / _D_IN)

    S = _dotT(xb, C)
    bmu = _bmu_feature(c2 - 2.0 * S, cmean, _M * _N)
    h_pre = jax.lax.dot_general(
        xb, w1_ref[0:_D_IN, :], (((1,), (0,)), ((), ())),
        preferred_element_type=jnp.float32)
    h = jax.nn.sigmoid(h_pre + bmu * w1_ref[_D_IN:_D_IN + 1, :] + b1_ref[...])
    hid_ref[...] = h

    ones_h = jnp.ones((1, _D_HID), jnp.float32)
    c2h = _dotT(ones_h, CH * CH)
    chmean = _dotT(ones_h, CH) * (1.0 / _D_HID)
    S2 = _dotT(h, CH)
    bmu2 = _bmu_feature(c2h - 2.0 * S2, chmean, _MH * _NH)
    o_pre = jax.lax.dot_general(
        h, w2_ref[0:_D_HID, :], (((1,), (0,)), ((), ())),
        preferred_element_type=jnp.float32)
    out_ref[...] = jax.nn.sigmoid(
        o_pre + bmu2 * w2_ref[_D_HID:_D_HID + 1, :] + b2_ref[...])


@functools.partial(jax.jit, static_argnames=())
def kernel(x, W1, b1, W2, b2, som_centroids, som_hidd_centroids):
    b1r = b1.reshape(1, _D_HID)
    b2r = b2.reshape(1, _D_OUT)
    grid = (_B // _TB,)
    const = lambda i: (0, 0)
    out, hid = pl.pallas_call(
        _emsom_kernel,
        grid=grid,
        in_specs=[
            pl.BlockSpec((_TB, _D_IN), lambda i: (i, 0)),
            pl.BlockSpec((_D_IN + 1, _D_HID), const),
            pl.BlockSpec((1, _D_HID), const),
            pl.BlockSpec((_D_HID + 1, _D_OUT), const),
            pl.BlockSpec((1, _D_OUT), const),
            pl.BlockSpec((_M * _N, _D_IN), const),
            pl.BlockSpec((_MH * _NH, _D_HID), const),
        ],
        out_specs=[
            pl.BlockSpec((_TB, _D_OUT), lambda i: (i, 0)),
            pl.BlockSpec((_TB, _D_HID), lambda i: (i, 0)),
        ],
        out_shape=[
            jax.ShapeDtypeStruct((_B, _D_OUT), jnp.float32),
            jax.ShapeDtypeStruct((_B, _D_HID), jnp.float32),
        ],
    )(x, W1, b1r, W2, b2r, som_centroids, som_hidd_centroids)
    return (out, hid)
